# baseline (speedup 1.0000x reference)
import jax
import jax.numpy as jnp
from jax import lax
from jax.experimental import pallas as pl
from jax.experimental.pallas import tpu as pltpu
from jax.experimental.pallas import tpu_sc as plsc

_B, _N, _Q = 16, 2048, 128
_HALF = _Q // 2
_NVREG = _HALF // 16
_GROUP = 16


def _sc_pool_body(xs_hbm, ys_hbm, samples_hbm, pooled_hbm, xs_v, ys_v, samp_v,
                  acc_v):
    c = lax.axis_index("c")
    s = lax.axis_index("s")
    wid = s * 2 + c
    b = wid // 2
    h = wid % 2
    pltpu.sync_copy(xs_hbm.at[b], xs_v)
    pltpu.sync_copy(ys_hbm.at[b], ys_v)
    pltpu.sync_copy(samples_hbm.at[h], samp_v)

    s2 = [samp_v[pl.ds(16 * j, 16)] for j in range(_NVREG)]
    s2 = [v + v for v in s2]
    zero = jnp.zeros((16,), jnp.float32)

    def body(g, accs):
        accs = list(accs)
        xv = xs_v[pl.ds(g * _GROUP, _GROUP)]
        yv = ys_v[pl.ds(g * _GROUP, _GROUP)]
        m2v = xv + yv
        perv = yv - xv
        for u in range(_GROUP):
            m2 = m2v[u]
            per = perv[u]
            for j in range(_NVREG):
                accs[j] = accs[j] + jnp.maximum(per - jnp.abs(s2[j] - m2), 0.0)
        return tuple(accs)

    accs = lax.fori_loop(0, _N // _GROUP, body, (zero,) * _NVREG)
    for j in range(_NVREG):
        acc_v[pl.ds(16 * j, 16)] = accs[j]
    pltpu.sync_copy(acc_v, pooled_hbm.at[b, h])


_sc_pool = pl.kernel(
    _sc_pool_body,
    out_type=jax.ShapeDtypeStruct((_B, 2, _HALF), jnp.float32),
    mesh=plsc.VectorSubcoreMesh(core_axis_name="c", subcore_axis_name="s"),
    scratch_types=[
        pltpu.VMEM((_N,), jnp.float32),
        pltpu.VMEM((_N,), jnp.float32),
        pltpu.VMEM((_HALF,), jnp.float32),
        pltpu.VMEM((_HALF,), jnp.float32),
    ],
)


def _tc_rho_body(pooled_ref, w_ref, b_ref, out_ref):
    pooled = pooled_ref[...] * 0.5
    acc = lax.dot_general(
        pooled, w_ref[...], (((1,), (0,)), ((), ())),
        preferred_element_type=jnp.float32,
    )
    out_ref[...] = jnp.maximum(acc + b_ref[...], 0.0)


_tc_rho = pl.pallas_call(
    _tc_rho_body,
    out_shape=jax.ShapeDtypeStruct((_B, _Q), jnp.float32),
)


def kernel(diagram, samples, rho_w, rho_b):
    xs = diagram[:, :, 0]
    ys = diagram[:, :, 1]
    pooled2 = _sc_pool(xs, ys, samples.reshape(2, _HALF))
    return _tc_rho(pooled2.reshape(_B, _Q), rho_w.T, rho_b.reshape(1, _Q))
